# Initial kernel scaffold; baseline (speedup 1.0000x reference)
import jax
import jax.numpy as jnp
from jax.experimental import pallas as pl


def kernel(dense_0, dense_1, dense_2, dense_3, dense_4, dense_5, dense_6, dense_7, dense_8, dense_9, dense_10, dense_11, dense_12, sparse_0, sparse_1, sparse_2, sparse_3, sparse_4, sparse_5, sparse_6, sparse_7, sparse_8, sparse_9, sparse_10, sparse_11, sparse_12, sparse_13, sparse_14, sparse_15, sparse_16, sparse_17, sparse_18, sparse_19, sparse_20, sparse_21, sparse_22, sparse_23, sparse_24, sparse_25, emb_tables, ln_gamma, ln_beta, W_lin, b_lin, W1, b1, W2, b2, W_out, b_out):
    raise NotImplementedError("write your pallas kernel here")


# speedup vs baseline: 1.0846x; 1.0846x over previous
import functools

import jax
import jax.numpy as jnp
from jax import lax
from jax.experimental import pallas as pl
from jax.experimental.pallas import tpu as pltpu
from jax.experimental.pallas import tpu_sc as plsc

B = 16384
ND = 13
NS = 26
V = 100000
D = 16
H1, H2 = 128, 64
IN_DIM = ND + NS * D

NCORES = 2
NSUB = 16
NWORK = NCORES * NSUB
GB = 128
NGROUPS = B // GB
GROUPS_PER_WORKER = NGROUPS // NWORK
GROWS = GB * NS

TILE_B = 2048


def _sc_gather_body(emb_hbm, idx_hbm, out_hbm, idx_v, buf_v, sem):
    w = lax.axis_index("s") * NCORES + lax.axis_index("c")

    def one_group(t, carry):
        g = w * GROUPS_PER_WORKER + t
        pltpu.sync_copy(idx_hbm.at[pl.ds(g * GROWS, GROWS)], idx_v)

        def fire(k, c):
            pltpu.make_async_copy(
                emb_hbm.at[idx_v.at[pl.ds(k * GB, GB)]],
                buf_v.at[pl.ds(k * GB, GB)],
                sem,
            ).start()
            return c

        def drain(k, c):
            pltpu.make_async_copy(
                emb_hbm.at[idx_v.at[pl.ds(k * GB, GB)]],
                buf_v.at[pl.ds(k * GB, GB)],
                sem,
            ).wait()
            return c

        lax.fori_loop(0, NS, fire, 0)
        lax.fori_loop(0, NS, drain, 0)
        pltpu.sync_copy(buf_v, out_hbm.at[pl.ds(g * GROWS, GROWS)])
        return carry

    lax.fori_loop(0, GROUPS_PER_WORKER, one_group, 0)


@functools.lru_cache(maxsize=None)
def _sc_gather_kernel():
    return pl.kernel(
        _sc_gather_body,
        out_type=jax.ShapeDtypeStruct((NS * B, D), jnp.float32),
        mesh=plsc.VectorSubcoreMesh(core_axis_name="c", subcore_axis_name="s"),
        scratch_types=[
            pltpu.VMEM((GROWS,), jnp.int32),
            pltpu.VMEM((GROWS, D), jnp.float32),
            pltpu.SemaphoreType.DMA,
        ],
        compiler_params=pltpu.CompilerParams(use_tc_tiling_on_sc=False),
    )


def _tc_body(x_ref, beta_ref, wlin_ref, blin_ref, w1_ref, b1_ref, w2_ref,
             b2_ref, wout_ref, bout_ref, m_ref, o_ref):
    hi = jax.lax.Precision.HIGHEST
    x = x_ref[...]
    beta = beta_ref[...]
    w1 = w1_ref[...]
    wlin = wlin_ref[...]
    b1 = beta @ w1[:ND] + b1_ref[...]
    h1 = jnp.maximum(jnp.dot(x, w1[ND:], precision=hi) + b1, 0.0)
    h2 = jnp.maximum(jnp.dot(h1, w2_ref[...], precision=hi) + b2_ref[...], 0.0)
    lin = (jnp.dot(x, wlin[ND:], precision=hi)
           + beta @ wlin[:ND] + blin_ref[...])
    sm = jnp.dot(x, m_ref[...], precision=hi)
    cross = 0.5 * (jnp.sum(sm * sm, axis=-1, keepdims=True)
                   - jnp.sum(x * x, axis=-1, keepdims=True))
    wout = wout_ref[...]
    z = ((lin + cross) * jnp.sum(wout)
         + jnp.dot(h2, wout, precision=hi) + bout_ref[...])
    o_ref[...] = jax.nn.sigmoid(z)


def _tc_forward(x, beta, wlin, blin, w1, b1, w2, b2, wout, bout, msum):
    grid = (B // TILE_B,)
    full = lambda shape: pl.BlockSpec(shape, lambda g: (0, 0))
    return pl.pallas_call(
        _tc_body,
        grid=grid,
        in_specs=[
            pl.BlockSpec((TILE_B, NS * D), lambda g: (g, 0)),
            full((1, ND)),
            full((IN_DIM, 1)),
            full((1, 1)),
            full((IN_DIM, H1)),
            full((1, H1)),
            full((H1, H2)),
            full((1, H2)),
            full((H2, 1)),
            full((1, 1)),
            full((NS * D, D)),
        ],
        out_specs=pl.BlockSpec((TILE_B, 1), lambda g: (g, 0)),
        out_shape=jax.ShapeDtypeStruct((B, 1), jnp.float32),
    )(x, beta, wlin, blin, w1, b1, w2, b2, wout, bout, msum)


def kernel(dense_0, dense_1, dense_2, dense_3, dense_4, dense_5, dense_6,
           dense_7, dense_8, dense_9, dense_10, dense_11, dense_12,
           sparse_0, sparse_1, sparse_2, sparse_3, sparse_4, sparse_5,
           sparse_6, sparse_7, sparse_8, sparse_9, sparse_10, sparse_11,
           sparse_12, sparse_13, sparse_14, sparse_15, sparse_16, sparse_17,
           sparse_18, sparse_19, sparse_20, sparse_21, sparse_22, sparse_23,
           sparse_24, sparse_25,
           emb_tables, ln_gamma, ln_beta, W_lin, b_lin, W1, b1, W2, b2,
           W_out, b_out):
    sparse = [sparse_0, sparse_1, sparse_2, sparse_3, sparse_4, sparse_5,
              sparse_6, sparse_7, sparse_8, sparse_9, sparse_10, sparse_11,
              sparse_12, sparse_13, sparse_14, sparse_15, sparse_16,
              sparse_17, sparse_18, sparse_19, sparse_20, sparse_21,
              sparse_22, sparse_23, sparse_24, sparse_25]
    idx = jnp.concatenate(sparse, axis=1)
    idx_flat = (idx + jnp.arange(NS, dtype=jnp.int32)[None, :] * V).reshape(-1)
    emb_flat = emb_tables.reshape(NS * V, D)

    gathered = _sc_gather_kernel()(emb_flat, idx_flat)
    gathered = gathered.reshape(B, NS * D)

    msum = jnp.tile(jnp.eye(D, dtype=jnp.float32), (NS, 1))
    return _tc_forward(
        gathered,
        ln_beta.reshape(1, ND),
        W_lin,
        b_lin.reshape(1, 1),
        W1,
        b1.reshape(1, H1),
        W2,
        b2.reshape(1, H2),
        W_out,
        b_out.reshape(1, 1),
        msum,
    )
